# baseline (speedup 1.0000x reference)
import functools
import jax
import jax.numpy as jnp
from jax import lax
from jax.experimental import pallas as pl
from jax.experimental.pallas import tpu as pltpu
from jax.experimental.pallas import tpu_sc as plsc

_SEQ = 2048
_D = 256
_NC = 2
_NS = 16
_LANES = 16


def kernel(z, pos_embed):
    batch = z.shape[1]
    nw = _NC * _NS
    rows_per_w = _SEQ // nw
    nchunks = _D // _LANES

    mesh = plsc.VectorSubcoreMesh(core_axis_name="c", subcore_axis_name="s")

    @functools.partial(
        pl.kernel,
        mesh=mesh,
        out_type=jax.ShapeDtypeStruct((_SEQ * batch, _D), jnp.float32),
        scratch_types=[
            pltpu.VMEM((rows_per_w, _D), jnp.float32),
            pltpu.VMEM((batch, _D), jnp.float32),
        ],
    )
    def sc_fill(pe_hbm, out_hbm, tab_v, blk_v):
        wid = lax.axis_index("s") * _NC + lax.axis_index("c")
        base = wid * rows_per_w
        pltpu.sync_copy(pe_hbm.at[pl.ds(base, rows_per_w)], tab_v)

        def row_body(i, carry):
            vs = [tab_v[i, pl.ds(c * _LANES, _LANES)] for c in range(nchunks)]

            def j_body(jj, c2):
                for u in range(8):
                    j = jj * 8 + u
                    for c in range(nchunks):
                        blk_v[j, pl.ds(c * _LANES, _LANES)] = vs[c]
                return c2

            lax.fori_loop(0, batch // 8, j_body, 0)
            pltpu.sync_copy(blk_v, out_hbm.at[pl.ds((base + i) * batch, batch)])
            return carry

        lax.fori_loop(0, rows_per_w, row_body, 0)

    out = sc_fill(pos_embed)
    return out.reshape(_SEQ, batch, _D)


# speedup vs baseline: 1.6699x; 1.6699x over previous
import functools
import jax
import jax.numpy as jnp
from jax import lax
from jax.experimental import pallas as pl
from jax.experimental.pallas import tpu as pltpu
from jax.experimental.pallas import tpu_sc as plsc

_SEQ = 2048
_D = 256
_NC = 2
_NS = 16
_LANES = 16


def kernel(z, pos_embed):
    batch = z.shape[1]
    nw = _NC * _NS
    rows_per_w = _SEQ // nw
    nchunks = _D // _LANES

    mesh = plsc.VectorSubcoreMesh(core_axis_name="c", subcore_axis_name="s")

    @functools.partial(
        pl.kernel,
        mesh=mesh,
        out_type=jax.ShapeDtypeStruct((_SEQ * batch, _D), jnp.float32),
        scratch_types=[
            pltpu.VMEM((rows_per_w, _D), jnp.float32),
            pltpu.VMEM((batch, _D), jnp.float32),
            pltpu.VMEM((batch, _D), jnp.float32),
            pltpu.SemaphoreType.DMA,
            pltpu.SemaphoreType.DMA,
        ],
    )
    def sc_fill(pe_hbm, out_hbm, tab_v, blk0_v, blk1_v, sem0, sem1):
        wid = lax.axis_index("s") * _NC + lax.axis_index("c")
        base = wid * rows_per_w
        pltpu.sync_copy(pe_hbm.at[pl.ds(base, rows_per_w)], tab_v)
        bufs = (blk0_v, blk1_v)
        sems = (sem0, sem1)

        def build(i, blk_v):
            vs = [tab_v[i, pl.ds(c * _LANES, _LANES)] for c in range(nchunks)]

            def j_body(jj, c2):
                for u in range(8):
                    j = jj * 8 + u
                    for c in range(nchunks):
                        blk_v[j, pl.ds(c * _LANES, _LANES)] = vs[c]
                return c2

            lax.fori_loop(0, batch // 8, j_body, 0)

        def start(i, b):
            pltpu.async_copy(
                bufs[b], out_hbm.at[pl.ds((base + i) * batch, batch)], sems[b]
            )

        def drain(i, b):
            pltpu.make_async_copy(
                bufs[b], out_hbm.at[pl.ds((base + i) * batch, batch)], sems[b]
            ).wait()

        for b in range(2):
            build(b, bufs[b])
            start(b, b)

        def t_body(t, carry):
            for b in range(2):
                i = t * 2 + b
                drain(i, b)
                build(i, bufs[b])
                start(i, b)
            return carry

        lax.fori_loop(1, rows_per_w // 2, t_body, 0)
        for b in range(2):
            drain(rows_per_w - 2 + b, b)

    out = sc_fill(pos_embed)
    return out.reshape(_SEQ, batch, _D)
